# baseline (speedup 1.0000x reference)
import functools
import jax
import jax.numpy as jnp
from jax import lax
from jax.experimental import pallas as pl
from jax.experimental.pallas import tpu as pltpu
from jax.experimental.pallas import tpu_sc as plsc

_L = 16
_H = 512
_HP = _H + 4
_NW = 32
_RW = _H // _NW
_RB = _RW + 4
_NTBL = 12 * 4096 * 2
_BIAS = 2048
_BIAS12 = 12 * _BIAS


def _rot_off(d, r):
    dx, dy = d
    if r == 0:
        return (dx, dy)
    if r == 1:
        return (dy, -dx)
    if r == 2:
        return (-dx, -dy)
    return (-dy, dx)


_BASE_OFF = {0: ((0, 1), (0, 2)), 1: ((1, 1), (2, 2)), 2: ((1, 2), (2, 1))}
_PERMS = {0: [0, 1, 2, 3], 1: [2, 0, 3, 1], 2: [3, 2, 1, 0], 3: [1, 3, 0, 2]}
_PASS_OFFS = [
    (_rot_off(_BASE_OFF[kt][0], r), _rot_off(_BASE_OFF[kt][1], r))
    for kt in range(3)
    for r in range(4)
]


def _body(p_hbm, t_hbm, out_hbm, raw_v, img_v, tbl_v, row_v, sem):
    wid = lax.axis_index("s") * 2 + lax.axis_index("c")
    tbl_cp = pltpu.async_copy(t_hbm, tbl_v, sem)

    g0 = wid * _RW - 2
    blk0 = lax.clamp(0, g0, _H - _RB)
    pltpu.sync_copy(p_hbm.at[pl.ds(blk0 * _H, _RB * _H)], raw_v)

    def prow(pr, carry):
        g = g0 + pr
        gr = (_H - 1) - jnp.abs((_H - 1) - jnp.abs(g))
        sbase = (gr - blk0) * _H
        dbase = pr * _HP + 2

        def pchunk(j, carry2):
            img_v[pl.ds(dbase + j * 16, 16)] = raw_v[pl.ds(sbase + j * 16, 16)]
            return carry2

        return lax.fori_loop(0, _H // 16, pchunk, carry)

    lax.fori_loop(0, _RB, prow, 0)

    for o in (0, _RB - 16):
        prv = lax.iota(jnp.int32, 16) + o
        g = g0 + prv
        grv = (_H - 1) - jnp.abs((_H - 1) - jnp.abs(g))
        srow = (grv - blk0) * _H
        drow = prv * _HP
        for dcol, scol in ((0, 2), (1, 1), (_HP - 2, _H - 2), (_HP - 1, _H - 3)):
            v = plsc.load_gather(raw_v, [srow + scol])
            plsc.store_scatter(img_v, [drow + dcol], v)

    tbl_cp.wait()

    io2 = lax.iota(jnp.int32, 16) * 2
    third = jnp.float32(1.0 / 3.0)
    bias = jnp.float32(_BIAS12)

    def do_chunk(k, carry):
        def row_body(i, carry):
            li = k * 4 + i

            def vec_body(j0, carry2):
                c0 = j0 * 16
                base2 = (li + 2) * _HP + c0 + 2

                def ld(dx, dy):
                    return img_v[pl.ds(base2 + (dx * _HP + dy), 16)]

                a8 = ld(0, 0) << 8
                acc_a = jnp.zeros((16,), jnp.int32)
                acc_b = jnp.zeros((16,), jnp.int32)
                for p, (vb, vc) in enumerate(_PASS_OFFS):
                    bv4 = ld(vb[0], vb[1]) << 4
                    cv = ld(vc[0], vc[1])
                    w0 = a8 + bv4 + cv
                    acc_a = acc_a + plsc.load_gather(
                        tbl_v.at[pl.ds(p * 8192, 4096)], [w0]
                    )
                    acc_b = acc_b + plsc.load_gather(
                        tbl_v.at[pl.ds(p * 8192 + 4096, 4096)], [w0]
                    )

                def fin(acc_half):
                    return (acc_half.astype(jnp.float32) - bias) * third

                ra = jnp.zeros((16,), jnp.int32) + 2 * i
                ci = 2 * c0 + io2
                plsc.store_scatter(row_v, [ra, ci], fin(acc_a & 0xFFFF))
                plsc.store_scatter(
                    row_v, [ra, ci + 1], fin(lax.shift_right_logical(acc_a, 16))
                )
                plsc.store_scatter(row_v, [ra + 1, ci], fin(acc_b & 0xFFFF))
                plsc.store_scatter(
                    row_v, [ra + 1, ci + 1], fin(lax.shift_right_logical(acc_b, 16))
                )
                return carry2

            return lax.fori_loop(0, 32, vec_body, carry)

        lax.fori_loop(0, 4, row_body, 0)
        pltpu.sync_copy(row_v, out_hbm.at[pl.ds(wid * 32 + k * 8, 8)])
        return carry

    lax.fori_loop(0, 4, do_chunk, 0)


@jax.jit
def kernel(img_lr, msb_weight):
    flat = img_lr.astype(jnp.int32).reshape(-1)

    w_int = msb_weight.astype(jnp.int32)
    los, his = [], []
    for r in range(4):
        p = _PERMS[r]
        los.append(
            (w_int[:, :, p[0]] + _BIAS) | ((w_int[:, :, p[1]] + _BIAS) << 16)
        )
        his.append(
            (w_int[:, :, p[2]] + _BIAS) | ((w_int[:, :, p[3]] + _BIAS) << 16)
        )
    lo = jnp.stack(los, axis=-1)
    hi = jnp.stack(his, axis=-1)
    table = jnp.stack([lo, hi], axis=-1).transpose(0, 2, 3, 1).reshape(-1)

    mesh = plsc.VectorSubcoreMesh(core_axis_name="c", subcore_axis_name="s")
    run = functools.partial(
        pl.kernel,
        mesh=mesh,
        compiler_params=pltpu.CompilerParams(needs_layout_passes=False),
        out_type=jax.ShapeDtypeStruct((2 * _H, 2 * _H), jnp.float32),
        scratch_types=[
            pltpu.VMEM((_RB * _H,), jnp.int32),
            pltpu.VMEM((_RB * _HP,), jnp.int32),
            pltpu.VMEM((_NTBL,), jnp.int32),
            pltpu.VMEM((8, 2 * _H), jnp.float32),
            pltpu.SemaphoreType.DMA,
        ],
    )(_body)
    return run(flat, table)


# speedup vs baseline: 1.0157x; 1.0157x over previous
import functools
import jax
import jax.numpy as jnp
from jax import lax
from jax.experimental import pallas as pl
from jax.experimental.pallas import tpu as pltpu
from jax.experimental.pallas import tpu_sc as plsc

_L = 16
_H = 512
_HP = _H + 4
_NW = 32
_RW = _H // _NW
_RB = _RW + 4
_NTBL = 12 * 4096 * 2
_BIAS = 2048
_BIAS12 = 12 * _BIAS


def _rot_off(d, r):
    dx, dy = d
    if r == 0:
        return (dx, dy)
    if r == 1:
        return (dy, -dx)
    if r == 2:
        return (-dx, -dy)
    return (-dy, dx)


_BASE_OFF = {0: ((0, 1), (0, 2)), 1: ((1, 1), (2, 2)), 2: ((1, 2), (2, 1))}
_PERMS = {0: [0, 1, 2, 3], 1: [2, 0, 3, 1], 2: [3, 2, 1, 0], 3: [1, 3, 0, 2]}
_PASS_OFFS = [
    (_rot_off(_BASE_OFF[kt][0], r), _rot_off(_BASE_OFF[kt][1], r))
    for kt in range(3)
    for r in range(4)
]


def _body(p_hbm, t_hbm, out_hbm, raw_v, img_v, tbl_v, row0_v, row1_v, sem, osem0, osem1):
    wid = lax.axis_index("s") * 2 + lax.axis_index("c")
    tbl_cp = pltpu.async_copy(t_hbm, tbl_v, sem)

    g0 = wid * _RW - 2
    blk0 = lax.clamp(0, g0, _H - _RB)
    pltpu.sync_copy(p_hbm.at[pl.ds(blk0 * _H, _RB * _H)], raw_v)

    def prow(pr, carry):
        g = g0 + pr
        gr = (_H - 1) - jnp.abs((_H - 1) - jnp.abs(g))
        sbase = (gr - blk0) * _H
        dbase = pr * _HP + 2

        def pchunk(j, carry2):
            img_v[pl.ds(dbase + j * 16, 16)] = raw_v[pl.ds(sbase + j * 16, 16)]
            return carry2

        return lax.fori_loop(0, _H // 16, pchunk, carry)

    lax.fori_loop(0, _RB, prow, 0)

    for o in (0, _RB - 16):
        prv = lax.iota(jnp.int32, 16) + o
        g = g0 + prv
        grv = (_H - 1) - jnp.abs((_H - 1) - jnp.abs(g))
        srow = (grv - blk0) * _H
        drow = prv * _HP
        for dcol, scol in ((0, 2), (1, 1), (_HP - 2, _H - 2), (_HP - 1, _H - 3)):
            v = plsc.load_gather(raw_v, [srow + scol])
            plsc.store_scatter(img_v, [drow + dcol], v)

    tbl_cp.wait()

    io2 = lax.iota(jnp.int32, 16) * 2
    third = jnp.float32(1.0 / 3.0)
    bias = jnp.float32(_BIAS12)

    def do_chunk(k, row_v):
        def row_body(i, carry):
            li = k * 2 + i

            def vec_body(j0, carry2):
                c0 = j0 * 16
                base2 = (li + 2) * _HP + c0 + 2

                def ld(dx, dy):
                    return img_v[pl.ds(base2 + (dx * _HP + dy), 16)]

                a8 = ld(0, 0) << 8
                acc_a = jnp.zeros((16,), jnp.int32)
                acc_b = jnp.zeros((16,), jnp.int32)
                for p, (vb, vc) in enumerate(_PASS_OFFS):
                    bv4 = ld(vb[0], vb[1]) << 4
                    cv = ld(vc[0], vc[1])
                    w0 = a8 + bv4 + cv + (p * 8192)
                    acc_a = acc_a + plsc.load_gather(tbl_v, [w0])
                    acc_b = acc_b + plsc.load_gather(tbl_v, [w0 + 4096])

                def fin(acc_half):
                    return (acc_half.astype(jnp.float32) - bias) * third

                ra = jnp.zeros((16,), jnp.int32) + 2 * i
                ci = 2 * c0 + io2
                plsc.store_scatter(row_v, [ra, ci], fin(acc_a & 0xFFFF))
                plsc.store_scatter(
                    row_v, [ra, ci + 1], fin(lax.shift_right_logical(acc_a, 16))
                )
                plsc.store_scatter(row_v, [ra + 1, ci], fin(acc_b & 0xFFFF))
                plsc.store_scatter(
                    row_v, [ra + 1, ci + 1], fin(lax.shift_right_logical(acc_b, 16))
                )
                return carry2

            return lax.fori_loop(0, 32, vec_body, carry)

        lax.fori_loop(0, 2, row_body, 0)

    bufs = (row0_v, row1_v)
    sems = (osem0, osem1)
    handles = [None, None]
    for k in range(8):
        b = k & 1
        if handles[b] is not None:
            handles[b].wait()
        do_chunk(k, bufs[b])
        handles[b] = pltpu.async_copy(
            bufs[b], out_hbm.at[pl.ds(wid * 32 + k * 4, 4)], sems[b]
        )
    handles[0].wait()
    handles[1].wait()


@jax.jit
def kernel(img_lr, msb_weight):
    flat = img_lr.astype(jnp.int32).reshape(-1)

    w_int = msb_weight.astype(jnp.int32)
    los, his = [], []
    for r in range(4):
        p = _PERMS[r]
        los.append(
            (w_int[:, :, p[0]] + _BIAS) | ((w_int[:, :, p[1]] + _BIAS) << 16)
        )
        his.append(
            (w_int[:, :, p[2]] + _BIAS) | ((w_int[:, :, p[3]] + _BIAS) << 16)
        )
    lo = jnp.stack(los, axis=-1)
    hi = jnp.stack(his, axis=-1)
    table = jnp.stack([lo, hi], axis=-1).transpose(0, 2, 3, 1).reshape(-1)

    mesh = plsc.VectorSubcoreMesh(core_axis_name="c", subcore_axis_name="s")
    run = functools.partial(
        pl.kernel,
        mesh=mesh,
        compiler_params=pltpu.CompilerParams(needs_layout_passes=False),
        out_type=jax.ShapeDtypeStruct((2 * _H, 2 * _H), jnp.float32),
        scratch_types=[
            pltpu.VMEM((_RB * _H,), jnp.int32),
            pltpu.VMEM((_RB * _HP,), jnp.int32),
            pltpu.VMEM((_NTBL,), jnp.int32),
            pltpu.VMEM((4, 2 * _H), jnp.float32),
            pltpu.VMEM((4, 2 * _H), jnp.float32),
            pltpu.SemaphoreType.DMA,
            pltpu.SemaphoreType.DMA,
            pltpu.SemaphoreType.DMA,
        ],
    )(_body)
    return run(flat, table)
